# baseline (speedup 1.0000x reference)
import jax
import jax.numpy as jnp
from jax import lax
from jax.experimental import pallas as pl
from jax.experimental.pallas import tpu as pltpu
from jax.experimental.pallas import tpu_sc as plsc

N = 10000
D = 128
H = 128
C = 40

NC = 2
NS = 16
NW = NC * NS
BATCH = 128
N_PAD = 10240
DUMP = N
RPT = N_PAD // NS

_PREC = lax.Precision.HIGHEST


def _proj_body(x_ref, wl_ref, wr_ref, b_ref, pa_ref, pb_ref, sa_ref, sb_ref):
    xb = x_ref[...]
    p = jnp.dot(xb, wl_ref[...], precision=_PREC)
    s = jnp.dot(xb, wr_ref[...], precision=_PREC) + b_ref[...]
    pa_ref[...] = p[:, :64]
    pb_ref[...] = p[:, 64:]
    sa_ref[...] = s[:, :64]
    sb_ref[...] = s[:, 64:]


def _mid_body(acca_ref, accb_ref, wl_ref, wr_ref, b_ref, pa_ref, pb_ref, s_ref):
    h = jnp.maximum(jnp.concatenate([acca_ref[0], accb_ref[0]], axis=1), 0.0)
    p = jnp.dot(h, wl_ref[...], precision=_PREC)
    pa_ref[...] = p
    pb_ref[...] = p
    s_ref[...] = 0.5 * (jnp.dot(h, wr_ref[...], precision=_PREC) + b_ref[...])


def _final_body(acc_ref, o_ref):
    o_ref[...] = acc_ref[0] + acc_ref[1]


NBUF = 2


def _edge_loop(p_hbm, src_v, dst_v, acc_sh, bufs, gsems, T):
    def step(i, carry):
        j0 = 2 * i
        j1 = j0 + 1
        d0 = pltpu.async_copy(p_hbm.at[src_v.at[j0]], bufs[0], gsems[0])
        d1 = pltpu.async_copy(p_hbm.at[src_v.at[j1]], bufs[1], gsems[1])
        d0.wait()
        pltpu.sync_copy(bufs[0], acc_sh.at[dst_v.at[j0]], add=True)
        d1.wait()
        pltpu.sync_copy(bufs[1], acc_sh.at[dst_v.at[j1]], add=True)
        return carry

    lax.fori_loop(0, T // 2, step, 0)


def _make_sc_l1(T2):
    mesh = plsc.VectorSubcoreMesh(core_axis_name="c", subcore_axis_name="s")

    def body(pa_hbm, pb_hbm, sa_hbm, sb_hbm, src_hbm, dst_hbm, out_hbm,
             src_v, dst_v, bufs, acc_sh, gsems):
        c = lax.axis_index("c")
        s = lax.axis_index("s")
        r0 = s * RPT
        pltpu.sync_copy(src_hbm.at[pl.ds(2 * s, 2)], src_v)
        pltpu.sync_copy(dst_hbm.at[pl.ds(2 * s, 2)], dst_v)

        @pl.when(c == 0)
        def _():
            pltpu.sync_copy(sa_hbm.at[pl.ds(r0, RPT)], acc_sh.at[pl.ds(r0, RPT)])

        @pl.when(c == 1)
        def _():
            pltpu.sync_copy(sb_hbm.at[pl.ds(r0, RPT)], acc_sh.at[pl.ds(r0, RPT)])

        plsc.subcore_barrier()

        @pl.when(c == 0)
        def _():
            _edge_loop(pa_hbm, src_v.at[0], dst_v.at[0], acc_sh, bufs, gsems, T2)
            _edge_loop(pa_hbm, src_v.at[1], dst_v.at[1], acc_sh, bufs, gsems, T2)

        @pl.when(c == 1)
        def _():
            _edge_loop(pb_hbm, src_v.at[0], dst_v.at[0], acc_sh, bufs, gsems, T2)
            _edge_loop(pb_hbm, src_v.at[1], dst_v.at[1], acc_sh, bufs, gsems, T2)

        plsc.subcore_barrier()
        pltpu.sync_copy(acc_sh.at[pl.ds(r0, RPT)], out_hbm.at[c, pl.ds(r0, RPT)])

    return pl.kernel(
        body,
        out_type=jax.ShapeDtypeStruct((NC, N_PAD, 64), jnp.float32),
        mesh=mesh,
        compiler_params=pltpu.CompilerParams(use_tc_tiling_on_sc=False),
        scratch_types=[
            pltpu.VMEM((2, T2, BATCH), jnp.int32),
            pltpu.VMEM((2, T2, BATCH), jnp.int32),
            [pltpu.VMEM((BATCH, 64), jnp.float32) for _ in range(NBUF)],
            pltpu.VMEM_SHARED((N_PAD, 64), jnp.float32),
            [pltpu.SemaphoreType.DMA for _ in range(NBUF)],
        ],
    )


def _make_sc_l2(T):
    mesh = plsc.VectorSubcoreMesh(core_axis_name="c", subcore_axis_name="s")

    def body(pa_hbm, pb_hbm, sh_hbm, src_hbm, dst_hbm, out_hbm,
             src_v, dst_v, bufs, acc_sh, gsems):
        c = lax.axis_index("c")
        s = lax.axis_index("s")
        w = c * NS + s
        r0 = s * RPT
        pltpu.sync_copy(sh_hbm.at[pl.ds(r0, RPT)], acc_sh.at[pl.ds(r0, RPT)])
        pltpu.sync_copy(src_hbm.at[w], src_v)
        pltpu.sync_copy(dst_hbm.at[w], dst_v)
        plsc.subcore_barrier()

        @pl.when(c == 0)
        def _():
            _edge_loop(pa_hbm, src_v, dst_v, acc_sh, bufs, gsems, T)

        @pl.when(c == 1)
        def _():
            _edge_loop(pb_hbm, src_v, dst_v, acc_sh, bufs, gsems, T)

        plsc.subcore_barrier()
        pltpu.sync_copy(acc_sh.at[pl.ds(r0, RPT)], out_hbm.at[c, pl.ds(r0, RPT)])

    return pl.kernel(
        body,
        out_type=jax.ShapeDtypeStruct((NC, N_PAD, C), jnp.float32),
        mesh=mesh,
        compiler_params=pltpu.CompilerParams(use_tc_tiling_on_sc=False),
        scratch_types=[
            pltpu.VMEM((T, BATCH), jnp.int32),
            pltpu.VMEM((T, BATCH), jnp.int32),
            [pltpu.VMEM((BATCH, C), jnp.float32) for _ in range(NBUF)],
            pltpu.VMEM_SHARED((N_PAD, C), jnp.float32),
            [pltpu.SemaphoreType.DMA for _ in range(NBUF)],
        ],
    )


def kernel(x, edge_index, W1_l, b1, W1_r, W2_l, b2, W2_r):
    src = edge_index[0]
    dst = edge_index[1]
    E = src.shape[0]

    T2 = -(-E // (NW * BATCH))
    T2 += T2 % 2
    pad2 = T2 * NW * BATCH - E
    src2 = jnp.concatenate([src, jnp.zeros((pad2,), jnp.int32)]).reshape(NW, T2, BATCH)
    dst2 = jnp.concatenate([dst, jnp.full((pad2,), DUMP, jnp.int32)]).reshape(NW, T2, BATCH)

    p1a, p1b, s1a, s1b = pl.pallas_call(
        _proj_body,
        grid=(25,),
        in_specs=[
            pl.BlockSpec((400, D), lambda i: (i, 0)),
            pl.BlockSpec((D, H), lambda i: (0, 0)),
            pl.BlockSpec((D, H), lambda i: (0, 0)),
            pl.BlockSpec((1, H), lambda i: (0, 0)),
        ],
        out_specs=[pl.BlockSpec((400, 64), lambda i: (i, 0)) for _ in range(4)],
        out_shape=[jax.ShapeDtypeStruct((N_PAD, 64), jnp.float32) for _ in range(4)],
    )(x, W1_l, W1_r, b1.reshape(1, H))

    acc1 = _make_sc_l1(T2)(p1a, p1b, s1a, s1b, src2, dst2)

    p2a, p2b, s2h = pl.pallas_call(
        _mid_body,
        grid=(8,),
        in_specs=[
            pl.BlockSpec((1, 1280, 64), lambda i: (0, i, 0)),
            pl.BlockSpec((1, 1280, 64), lambda i: (1, i, 0)),
            pl.BlockSpec((H, C), lambda i: (0, 0)),
            pl.BlockSpec((H, C), lambda i: (0, 0)),
            pl.BlockSpec((1, C), lambda i: (0, 0)),
        ],
        out_specs=[pl.BlockSpec((1280, C), lambda i: (i, 0)) for _ in range(3)],
        out_shape=[jax.ShapeDtypeStruct((N_PAD, C), jnp.float32) for _ in range(3)],
    )(acc1, acc1, W2_l, W2_r, b2.reshape(1, C))

    acc2 = _make_sc_l2(T2)(p2a, p2b, s2h, src2, dst2)

    out = pl.pallas_call(
        _final_body,
        grid=(5,),
        in_specs=[pl.BlockSpec((NC, 2000, C), lambda i: (0, i, 0))],
        out_specs=pl.BlockSpec((2000, C), lambda i: (i, 0)),
        out_shape=jax.ShapeDtypeStruct((N, C), jnp.float32),
    )(acc2)
    return out
